# baseline (device time: 109360 ns/iter reference)
import jax
import jax.numpy as jnp
from jax import lax
from jax.experimental import pallas as pl
from jax.experimental.pallas import tpu as pltpu

N_DEV = 8
N_EXP = 16
CAP = 128
ROWS = 2 * CAP


def _a2a_moe_pallas(xb, a_col, a_row, w1b, w2b):
    t, d = xb.shape

    def body(xb_ref, a_col_ref, a_row_ref, w1_ref, w2_ref, final_ref,
             xsend_ref, xrecv_ref, y_ref, outb_ref,
             send1, recv1, send2, recv2):
        my = lax.axis_index("i")

        bsem = pltpu.get_barrier_semaphore()
        for o in range(1, N_DEV):
            pl.semaphore_signal(
                bsem, inc=1,
                device_id=((my + o) % N_DEV,),
                device_id_type=pl.DeviceIdType.MESH,
            )

        ac = a_col_ref[...]
        ar = a_row_ref[...]
        row_i = lax.broadcasted_iota(jnp.int32, (t, t), 0)
        col_i = lax.broadcasted_iota(jnp.int32, (t, t), 1)
        eq = ac == ar
        c_col = jnp.sum(jnp.where(eq & (col_i < row_i), 1.0, 0.0),
                        axis=1, keepdims=True)
        c_row = jnp.sum(jnp.where(eq & (row_i < col_i), 1.0, 0.0),
                        axis=0, keepdims=True)
        rm_col = (ac >> 1) * ROWS + (ac & 1) * CAP + c_col.astype(jnp.int32)
        rm_row = (ar >> 1) * ROWS + (ar & 1) * CAP + c_row.astype(jnp.int32)
        pt = jnp.where(
            rm_col == lax.broadcasted_iota(jnp.int32, (t, N_DEV * ROWS), 1),
            1.0, 0.0).astype(jnp.bfloat16)
        pr = jnp.where(
            rm_row == lax.broadcasted_iota(jnp.int32, (N_DEV * ROWS, t), 0),
            1.0, 0.0).astype(jnp.bfloat16)

        xsend_flat = jnp.dot(
            pr, xb_ref[...], preferred_element_type=jnp.float32,
        ).astype(jnp.bfloat16)
        xsend_ref[...] = xsend_flat.reshape(N_DEV, ROWS, d)

        pl.semaphore_wait(bsem, N_DEV - 1)

        pltpu.make_async_copy(
            xsend_ref.at[my], xrecv_ref.at[my], recv1.at[my]
        ).start()
        for o in range(1, N_DEV):
            dst = (my + o) % N_DEV
            pltpu.make_async_remote_copy(
                src_ref=xsend_ref.at[dst],
                dst_ref=xrecv_ref.at[my],
                send_sem=send1.at[dst],
                recv_sem=recv1.at[my],
                device_id=(dst,),
                device_id_type=pl.DeviceIdType.MESH,
            ).start()
        for o in range(N_DEV):
            s = (my + o) % N_DEV
            pltpu.make_async_copy(
                xrecv_ref.at[s], xrecv_ref.at[s], recv1.at[s]
            ).wait()
            for half in range(2):
                lo = half * CAP
                xh = xrecv_ref[s, lo:lo + CAP, :]
                h = jnp.maximum(
                    jnp.dot(xh, w1_ref[half],
                            preferred_element_type=jnp.float32),
                    0.0).astype(jnp.bfloat16)
                yh = jnp.dot(h, w2_ref[half],
                             preferred_element_type=jnp.float32)
                y_ref[s, lo:lo + CAP, :] = yh.astype(jnp.bfloat16)
            if o == 0:
                pltpu.make_async_copy(
                    y_ref.at[my], outb_ref.at[my], recv2.at[my]
                ).start()
            else:
                pltpu.make_async_remote_copy(
                    src_ref=y_ref.at[s],
                    dst_ref=outb_ref.at[my],
                    send_sem=send2.at[s],
                    recv_sem=recv2.at[my],
                    device_id=(s,),
                    device_id_type=pl.DeviceIdType.MESH,
                ).start()

        for s in range(N_DEV):
            pltpu.make_async_copy(
                outb_ref.at[s], outb_ref.at[s], recv2.at[s]
            ).wait()

        final_ref[...] = jnp.dot(
            pt, outb_ref[...].reshape(N_DEV * ROWS, d),
            preferred_element_type=jnp.float32,
        ).astype(jnp.bfloat16)

        for o in range(1, N_DEV):
            dst = (my + o) % N_DEV
            pltpu.make_async_copy(
                xsend_ref.at[dst], xsend_ref.at[dst], send1.at[dst]
            ).wait()
            pltpu.make_async_copy(
                y_ref.at[dst], y_ref.at[dst], send2.at[dst]
            ).wait()

    return pl.pallas_call(
        body,
        out_shape=jax.ShapeDtypeStruct((t, d), jnp.bfloat16),
        in_specs=[
            pl.BlockSpec(memory_space=pltpu.VMEM),
            pl.BlockSpec(memory_space=pltpu.VMEM),
            pl.BlockSpec(memory_space=pltpu.VMEM),
            pl.BlockSpec(memory_space=pltpu.VMEM),
            pl.BlockSpec(memory_space=pltpu.VMEM),
        ],
        out_specs=pl.BlockSpec(memory_space=pltpu.VMEM),
        scratch_shapes=[
            pltpu.VMEM((N_DEV, ROWS, d), jnp.bfloat16),
            pltpu.VMEM((N_DEV, ROWS, d), jnp.bfloat16),
            pltpu.VMEM((N_DEV, ROWS, d), jnp.bfloat16),
            pltpu.VMEM((N_DEV, ROWS, d), jnp.bfloat16),
            pltpu.SemaphoreType.DMA((N_DEV,)),
            pltpu.SemaphoreType.DMA((N_DEV,)),
            pltpu.SemaphoreType.DMA((N_DEV,)),
            pltpu.SemaphoreType.DMA((N_DEV,)),
        ],
        compiler_params=pltpu.CompilerParams(
            collective_id=0, vmem_limit_bytes=100 * 1024 * 1024),
    )(xb, a_col, a_row, w1b, w2b)


def kernel(x, assign, W1, W2):
    t = assign.shape[0]
    out_b = _a2a_moe_pallas(
        x.astype(jnp.bfloat16),
        assign.reshape(t, 1),
        assign.reshape(1, t),
        W1.astype(jnp.bfloat16),
        W2.astype(jnp.bfloat16),
    )
    return out_b.astype(jnp.float32)


# device time: 104531 ns/iter; 1.0462x vs baseline; 1.0462x over previous
import jax
import jax.numpy as jnp
from jax import lax
from jax.experimental import pallas as pl
from jax.experimental.pallas import tpu as pltpu

N_DEV = 8
N_EXP = 16
CAP = 128
ROWS = 2 * CAP


def _a2a_moe_pallas(x, a_col, a_row, w1b, w2b):
    t, d = x.shape

    def body(x_ref, a_col_ref, a_row_ref, w1_ref, w2_ref, final_ref,
             xsend_ref, xrecv_ref, y_ref, outb_ref,
             send1, recv1, send2, recv2):
        my = lax.axis_index("i")

        bsem = pltpu.get_barrier_semaphore()
        for o in range(1, N_DEV):
            pl.semaphore_signal(
                bsem, inc=1,
                device_id=((my + o) % N_DEV,),
                device_id_type=pl.DeviceIdType.MESH,
            )

        ac = a_col_ref[...]
        ar = a_row_ref[...]
        row_i = lax.broadcasted_iota(jnp.int32, (t, t), 0)
        col_i = lax.broadcasted_iota(jnp.int32, (t, t), 1)
        eq = ac == ar
        c_col = jnp.sum(jnp.where(eq & (col_i < row_i), 1.0, 0.0),
                        axis=1, keepdims=True)
        c_row = jnp.sum(jnp.where(eq & (row_i < col_i), 1.0, 0.0),
                        axis=0, keepdims=True)
        rm_col = (ac >> 1) * ROWS + (ac & 1) * CAP + c_col.astype(jnp.int32)
        rm_row = (ar >> 1) * ROWS + (ar & 1) * CAP + c_row.astype(jnp.int32)

        xbv = x_ref[...].astype(jnp.bfloat16)

        pl.semaphore_wait(bsem, N_DEV - 1)

        slot_i = lax.broadcasted_iota(jnp.int32, (ROWS, t), 0)
        for dst in range(N_DEV):
            oh = jnp.where(rm_row == slot_i + dst * ROWS,
                           1.0, 0.0).astype(jnp.bfloat16)
            xsend_ref[dst] = jnp.dot(
                oh, xbv, preferred_element_type=jnp.float32,
            ).astype(jnp.bfloat16)

            @pl.when(my == dst)
            def _():
                pltpu.make_async_copy(
                    xsend_ref.at[dst], xrecv_ref.at[dst], recv1.at[dst]
                ).start()

            @pl.when(my != dst)
            def _():
                pltpu.make_async_remote_copy(
                    src_ref=xsend_ref.at[dst],
                    dst_ref=xrecv_ref.at[my],
                    send_sem=send1.at[dst],
                    recv_sem=recv1.at[my],
                    device_id=(dst,),
                    device_id_type=pl.DeviceIdType.MESH,
                ).start()

        for o in range(N_DEV):
            s = (my + o) % N_DEV
            pltpu.make_async_copy(
                xrecv_ref.at[s], xrecv_ref.at[s], recv1.at[s]
            ).wait()
            for half in range(2):
                lo = half * CAP
                xh = xrecv_ref[s, lo:lo + CAP, :]
                h = jnp.maximum(
                    jnp.dot(xh, w1_ref[half],
                            preferred_element_type=jnp.float32),
                    0.0).astype(jnp.bfloat16)
                yh = jnp.dot(h, w2_ref[half],
                             preferred_element_type=jnp.float32)
                y_ref[s, lo:lo + CAP, :] = yh.astype(jnp.bfloat16)
            if o == 0:
                pltpu.make_async_copy(
                    y_ref.at[my], outb_ref.at[my], recv2.at[my]
                ).start()
            else:
                pltpu.make_async_remote_copy(
                    src_ref=y_ref.at[s],
                    dst_ref=outb_ref.at[my],
                    send_sem=send2.at[s],
                    recv_sem=recv2.at[my],
                    device_id=(s,),
                    device_id_type=pl.DeviceIdType.MESH,
                ).start()

        lane_i = lax.broadcasted_iota(jnp.int32, (t, ROWS), 1)
        acc = jnp.zeros((t, d), jnp.float32)
        for s in range(N_DEV):
            pltpu.make_async_copy(
                outb_ref.at[s], outb_ref.at[s], recv2.at[s]
            ).wait()
            oh = jnp.where(rm_col == lane_i + s * ROWS,
                           1.0, 0.0).astype(jnp.bfloat16)
            acc = acc + jnp.dot(
                oh, outb_ref[s], preferred_element_type=jnp.float32)
        final_ref[...] = acc

        for o in range(1, N_DEV):
            dst = (my + o) % N_DEV
            pltpu.make_async_copy(
                xsend_ref.at[dst], xsend_ref.at[dst], send1.at[dst]
            ).wait()
            pltpu.make_async_copy(
                y_ref.at[dst], y_ref.at[dst], send2.at[dst]
            ).wait()

    return pl.pallas_call(
        body,
        out_shape=jax.ShapeDtypeStruct((t, d), jnp.float32),
        in_specs=[
            pl.BlockSpec(memory_space=pltpu.VMEM),
            pl.BlockSpec(memory_space=pltpu.VMEM),
            pl.BlockSpec(memory_space=pltpu.VMEM),
            pl.BlockSpec(memory_space=pltpu.VMEM),
            pl.BlockSpec(memory_space=pltpu.VMEM),
        ],
        out_specs=pl.BlockSpec(memory_space=pltpu.VMEM),
        scratch_shapes=[
            pltpu.VMEM((N_DEV, ROWS, d), jnp.bfloat16),
            pltpu.VMEM((N_DEV, ROWS, d), jnp.bfloat16),
            pltpu.VMEM((N_DEV, ROWS, d), jnp.bfloat16),
            pltpu.VMEM((N_DEV, ROWS, d), jnp.bfloat16),
            pltpu.SemaphoreType.DMA((N_DEV,)),
            pltpu.SemaphoreType.DMA((N_DEV,)),
            pltpu.SemaphoreType.DMA((N_DEV,)),
            pltpu.SemaphoreType.DMA((N_DEV,)),
        ],
        compiler_params=pltpu.CompilerParams(
            collective_id=0, vmem_limit_bytes=100 * 1024 * 1024),
    )(x, a_col, a_row, w1b, w2b)


def kernel(x, assign, W1, W2):
    t = assign.shape[0]
    return _a2a_moe_pallas(
        x,
        assign.reshape(t, 1),
        assign.reshape(1, t),
        W1.astype(jnp.bfloat16),
        W2.astype(jnp.bfloat16),
    )
